# baseline (speedup 1.0000x reference)
import jax
import jax.numpy as jnp
from jax import lax
from jax.experimental import pallas as pl
from jax.experimental.pallas import tpu as pltpu
from jax.experimental.pallas import tpu_sc as plsc

N_NODES = 10000
N_EDGES = 320000
D = 128

NC = 2
NS = 16
NPS = N_NODES // NC
DUMMY = 5000

CH = 32
NCHUNK = 640
SUP = 8
GE = SUP * CH
EDGES_PER_W = NCHUNK * CH
ACC_ROWS = 5008
ROWS_PER_TILE = 320
HROWS_PER_TILE = 624


def _mm_body(x_ref, w_ref, b_ref, h_ref):
    h_ref[...] = (
        jnp.dot(x_ref[...], w_ref[...], preferred_element_type=jnp.float32)
        + b_ref[...]
    )


def _combine_body(p_ref, o_ref):
    o_ref[...] = jnp.maximum(
        jnp.concatenate([p_ref[0, :NPS], p_ref[1, :NPS]], axis=0), 0.0
    )


def _sc_agg_body(h_hbm, src_hbm, dst_hbm, out_hbm,
                 htab, acc, src1d, dst1d, dstix, buf0, buf1,
                 gsem0, gsem1, ssem0, ssem1):
    cid = lax.axis_index("c")
    sid = lax.axis_index("s")
    lo = cid * NPS

    hbase = sid * HROWS_PER_TILE
    pltpu.sync_copy(h_hbm.at[pl.ds(hbase, HROWS_PER_TILE)],
                    htab.at[pl.ds(hbase, HROWS_PER_TILE)])

    @pl.when(sid == 0)
    def _():
        r = NS * HROWS_PER_TILE
        pltpu.sync_copy(h_hbm.at[pl.ds(r, N_NODES - r)],
                        htab.at[pl.ds(r, N_NODES - r)])

    zero = jnp.zeros((16,), jnp.float32)

    def _zrow(i, carry):
        for k in range(D // 16):
            buf0[i, pl.ds(k * 16, 16)] = zero
        return carry

    lax.fori_loop(0, CH, _zrow, 0)
    base = jnp.minimum(sid * ROWS_PER_TILE, ACC_ROWS - ROWS_PER_TILE)
    for t in range(ROWS_PER_TILE // CH):
        pltpu.sync_copy(buf0, acc.at[pl.ds(base + t * CH, CH)])
    plsc.subcore_barrier()

    def _gather(j, b, gsem):
        return pltpu.async_copy(htab.at[src1d.at[pl.ds(j * CH, CH)]],
                                b, gsem)

    def _scatter(j, b, ssem):
        return pltpu.async_copy(b, acc.at[dstix.at[j]], ssem, add=True)

    def _super(i, carry):
        pltpu.sync_copy(src_hbm.at[sid, pl.ds(i * GE, GE)], src1d)
        pltpu.sync_copy(dst_hbm.at[sid, pl.ds(i * GE, GE)], dst1d)
        for k in range(GE // 16):
            d = dst1d[pl.ds(k * 16, 16)]
            loc = d - lo
            inr = jnp.logical_and(loc >= 0, loc < NPS)
            dstix[k // 2, pl.ds((k % 2) * 16, 16)] = jnp.where(
                inr, loc, DUMMY)
        for p in range(SUP // 2):
            j0 = 2 * p
            j1 = j0 + 1
            g0 = _gather(j0, buf0, gsem0)
            g1 = _gather(j1, buf1, gsem1)
            g0.wait()
            s0 = _scatter(j0, buf0, ssem0)
            g1.wait()
            s1 = _scatter(j1, buf1, ssem1)
            s0.wait()
            s1.wait()
        return carry

    lax.fori_loop(0, NCHUNK // SUP, _super, 0)
    plsc.subcore_barrier()

    pltpu.sync_copy(acc.at[pl.ds(base, ROWS_PER_TILE)],
                    out_hbm.at[cid, pl.ds(base, ROWS_PER_TILE)])


def kernel(x, edge_index, W_disc, b_disc):
    h = pl.pallas_call(
        _mm_body,
        out_shape=jax.ShapeDtypeStruct((N_NODES, D), jnp.float32),
    )(x, W_disc, b_disc.reshape(1, D))

    src = edge_index[0].reshape(NS, N_EDGES // NS)
    dst = edge_index[1].reshape(NS, N_EDGES // NS)
    pad = EDGES_PER_W - N_EDGES // NS
    src_p = jnp.concatenate([src, jnp.zeros((NS, pad), jnp.int32)], axis=1)
    dst_p = jnp.concatenate(
        [dst, jnp.full((NS, pad), N_NODES, jnp.int32)], axis=1)

    mesh = plsc.VectorSubcoreMesh(
        core_axis_name="c", subcore_axis_name="s",
        num_cores=NC, num_subcores=NS,
    )
    partial = pl.kernel(
        _sc_agg_body,
        out_type=jax.ShapeDtypeStruct((NC, ACC_ROWS, D), jnp.float32),
        mesh=mesh,
        scratch_types=[
            pltpu.VMEM_SHARED((N_NODES, D), jnp.float32),
            pltpu.VMEM_SHARED((ACC_ROWS, D), jnp.float32),
            pltpu.VMEM((GE,), jnp.int32),
            pltpu.VMEM((GE,), jnp.int32),
            pltpu.VMEM((SUP, CH), jnp.int32),
            pltpu.VMEM((CH, D), jnp.float32),
            pltpu.VMEM((CH, D), jnp.float32),
            pltpu.SemaphoreType.DMA,
            pltpu.SemaphoreType.DMA,
            pltpu.SemaphoreType.DMA,
            pltpu.SemaphoreType.DMA,
        ],
    )(h, src_p, dst_p)

    out = pl.pallas_call(
        _combine_body,
        out_shape=jax.ShapeDtypeStruct((N_NODES, D), jnp.float32),
    )(partial)
    return out


# speedup vs baseline: 1.3269x; 1.3269x over previous
import jax
import jax.numpy as jnp
from jax import lax
from jax.experimental import pallas as pl
from jax.experimental.pallas import tpu as pltpu
from jax.experimental.pallas import tpu_sc as plsc

N_NODES = 10000
N_EDGES = 320000
D = 128

NC = 2
NS = 16
NPS = N_NODES // NC
DUMMY = 5000

CH = 32
NCHUNK = 640
SUP = 8
GE = SUP * CH
EDGES_PER_W = NCHUNK * CH
ACC_ROWS = 5008
ROWS_PER_TILE = 320
HROWS_PER_TILE = 624


def _mm_body(x_ref, w_ref, b_ref, h_ref):
    h_ref[...] = (
        jnp.dot(x_ref[...], w_ref[...], preferred_element_type=jnp.float32)
        + b_ref[...]
    )


def _combine_body(p_ref, o_ref):
    o_ref[...] = jnp.maximum(
        jnp.concatenate([p_ref[0, :NPS], p_ref[1, :NPS]], axis=0), 0.0
    )


def _sc_agg_body(h_hbm, sd_hbm, out_hbm,
                 htab, acc, sdbuf, dstix, buf0, buf1,
                 isem, gsem0, gsem1, ssem0, ssem1):
    cid = lax.axis_index("c")
    sid = lax.axis_index("s")
    lo = cid * NPS

    hbase = sid * HROWS_PER_TILE
    pltpu.sync_copy(h_hbm.at[pl.ds(hbase, HROWS_PER_TILE)],
                    htab.at[pl.ds(hbase, HROWS_PER_TILE)])

    @pl.when(sid == 0)
    def _():
        r = NS * HROWS_PER_TILE
        pltpu.sync_copy(h_hbm.at[pl.ds(r, N_NODES - r)],
                        htab.at[pl.ds(r, N_NODES - r)])

    zero = jnp.zeros((16,), jnp.float32)

    def _zrow(i, carry):
        for k in range(D // 16):
            buf0[i, pl.ds(k * 16, 16)] = zero
        return carry

    lax.fori_loop(0, CH, _zrow, 0)
    base = jnp.minimum(sid * ROWS_PER_TILE, ACC_ROWS - ROWS_PER_TILE)
    for t in range(ROWS_PER_TILE // CH):
        pltpu.sync_copy(buf0, acc.at[pl.ds(base + t * CH, CH)])
    plsc.subcore_barrier()

    def _gather(j, b, gsem):
        return pltpu.async_copy(
            htab.at[sdbuf.at[j // 4, pl.ds((j % 4) * CH, CH)]], b, gsem)

    def _scatter(j, b, ssem):
        return pltpu.async_copy(b, acc.at[dstix.at[j]], ssem, add=True)

    def _stage(i):
        return pltpu.async_copy(sd_hbm.at[sid, i], sdbuf, isem)

    NG = NCHUNK // SUP
    bufs = (buf0, buf1)
    gsems = (gsem0, gsem1)
    ssems = (ssem0, ssem1)

    _stage(0)

    def _super(i, carry):
        pltpu.make_async_copy(sd_hbm.at[sid, 0], sdbuf, isem).wait()
        for k in range(GE // 16):
            d = sdbuf[2 + k // 8, pl.ds((k % 8) * 16, 16)]
            loc = d - lo
            inr = jnp.logical_and(loc >= 0, loc < NPS)
            dstix[k // 2, pl.ds((k % 2) * 16, 16)] = jnp.where(
                inr, loc, DUMMY)
        gd = {0: _gather(0, buf0, gsem0)}
        sd = {}
        for c in range(SUP):
            b = c % 2
            ob = 1 - b
            if c >= 1:
                sd[ob].wait()
            if c + 1 < SUP:
                gd[ob] = _gather(c + 1, bufs[ob], gsems[ob])
            gd[b].wait()
            if c == SUP - 1:
                _stage(jnp.minimum(i + 1, NG - 1))
            sd[b] = _scatter(c, bufs[b], ssems[b])
        sd[(SUP - 1) % 2].wait()
        return carry

    lax.fori_loop(0, NG, _super, 0)
    pltpu.make_async_copy(sd_hbm.at[sid, 0], sdbuf, isem).wait()
    plsc.subcore_barrier()

    pltpu.sync_copy(acc.at[pl.ds(base, ROWS_PER_TILE)],
                    out_hbm.at[cid, pl.ds(base, ROWS_PER_TILE)])


def kernel(x, edge_index, W_disc, b_disc):
    h = pl.pallas_call(
        _mm_body,
        out_shape=jax.ShapeDtypeStruct((N_NODES, D), jnp.float32),
    )(x, W_disc, b_disc.reshape(1, D))

    src = edge_index[0].reshape(NS, N_EDGES // NS)
    dst = edge_index[1].reshape(NS, N_EDGES // NS)
    pad = EDGES_PER_W - N_EDGES // NS
    src_p = jnp.concatenate([src, jnp.zeros((NS, pad), jnp.int32)], axis=1)
    dst_p = jnp.concatenate(
        [dst, jnp.full((NS, pad), N_NODES, jnp.int32)], axis=1)
    sd_p = jnp.concatenate(
        [src_p.reshape(NS, -1, GE), dst_p.reshape(NS, -1, GE)], axis=2
    ).reshape(NS, -1, 4, 128)

    mesh = plsc.VectorSubcoreMesh(
        core_axis_name="c", subcore_axis_name="s",
        num_cores=NC, num_subcores=NS,
    )
    partial = pl.kernel(
        _sc_agg_body,
        out_type=jax.ShapeDtypeStruct((NC, ACC_ROWS, D), jnp.float32),
        mesh=mesh,
        scratch_types=[
            pltpu.VMEM_SHARED((N_NODES, D), jnp.float32),
            pltpu.VMEM_SHARED((ACC_ROWS, D), jnp.float32),
            pltpu.VMEM((4, 128), jnp.int32),
            pltpu.VMEM((SUP, CH), jnp.int32),
            pltpu.VMEM((CH, D), jnp.float32),
            pltpu.VMEM((CH, D), jnp.float32),
            pltpu.SemaphoreType.DMA,
            pltpu.SemaphoreType.DMA,
            pltpu.SemaphoreType.DMA,
            pltpu.SemaphoreType.DMA,
            pltpu.SemaphoreType.DMA,
        ],
    )(h, sd_p)

    out = pl.pallas_call(
        _combine_body,
        out_shape=jax.ShapeDtypeStruct((N_NODES, D), jnp.float32),
    )(partial)
    return out
